# baseline (speedup 1.0000x reference)
import functools

import jax
import jax.numpy as jnp
from jax.experimental import pallas as pl
from jax.experimental.pallas import tpu as pltpu

_N = 1024
_NP = 8
_NL = _N // _NP
_BR = 8

_roll = pltpu.roll


def _lane_iota():
    return jax.lax.broadcasted_iota(jnp.int32, (1, _NL), 1)


def _sort_planes(planes):
    li = _lane_iota()

    def flip(pls, k):
        if k >= _N:
            return pls
        if k < _NP:
            return [(-p if (i & k) else p) for i, p in enumerate(pls)]
        sgn = jnp.where((li & (k // _NP)) == 0, 1.0, -1.0).astype(pls[0].dtype)
        return [p * sgn for p in pls]

    k = 2
    while k <= _N:
        planes = flip(planes, k)
        j = k // 2
        while j >= 1:
            if j < _NP:
                out = list(planes)
                for p in range(_NP):
                    if p & j:
                        continue
                    a, b = planes[p], planes[p | j]
                    out[p] = jnp.minimum(a, b)
                    out[p | j] = jnp.maximum(a, b)
                planes = out
            else:
                m = j // _NP
                upper = (li & m) != 0
                nxt = []
                for p in planes:
                    down = _roll(p, _NL - m, axis=1)
                    mn = jnp.minimum(p, down)
                    mx = jnp.maximum(p, down)
                    nxt.append(jnp.where(upper, _roll(mx, m, axis=1), mn))
                planes = nxt
            j //= 2
        planes = flip(planes, k)
        k *= 2
    return planes


def _body(x_ref, y_ref, o_ref):
    xs = _sort_planes([x_ref[:, _NL * p:_NL * (p + 1)] for p in range(_NP)])
    ys = _sort_planes([y_ref[:, _NL * p:_NL * (p + 1)] for p in range(_NP)])
    for p in range(_NP):
        d = ys[p] - xs[p]
        o_ref[p] = 4.0 * jax.nn.sigmoid(2.0 * d) * jax.nn.sigmoid(-2.0 * d)


@functools.partial(jax.jit, static_argnames=("interpret",))
def kernel(x, y, interpret=False):
    b, s, n = x.shape
    rows = b * s
    xf = x.reshape(rows, n)
    yf = y.reshape(rows, n)
    out = pl.pallas_call(
        _body,
        grid=(rows // _BR,),
        in_specs=[
            pl.BlockSpec((_BR, n), lambda i: (i, 0)),
            pl.BlockSpec((_BR, n), lambda i: (i, 0)),
        ],
        out_specs=pl.BlockSpec((_NP, _BR, _NL), lambda i: (0, i, 0)),
        out_shape=jax.ShapeDtypeStruct((_NP, rows, _NL), jnp.float32),
        interpret=interpret,
    )(xf, yf)
    return out.transpose(1, 2, 0).reshape(b, s, n)


# speedup vs baseline: 3.3315x; 3.3315x over previous
import functools

import jax
import jax.numpy as jnp
from jax.experimental import pallas as pl
from jax.experimental.pallas import tpu as pltpu

_N = 1024
_NP = 8
_NL = _N // _NP
_BR = 32

_roll = pltpu.roll


def _lane_iota():
    return jax.lax.broadcasted_iota(jnp.int32, (1, _NL), 1)


def _sort_planes(planes):
    li = _lane_iota()

    def flip(pls, k):
        if k >= _N:
            return pls
        if k < _NP:
            return [(-p if (i & k) else p) for i, p in enumerate(pls)]
        sgn = jnp.where((li & (k // _NP)) == 0, 1.0, -1.0).astype(pls[0].dtype)
        return [p * sgn for p in pls]

    k = 2
    while k <= _N:
        planes = flip(planes, k)
        j = k // 2
        while j >= 1:
            if j < _NP:
                out = list(planes)
                for p in range(_NP):
                    if p & j:
                        continue
                    a, b = planes[p], planes[p | j]
                    out[p] = jnp.minimum(a, b)
                    out[p | j] = jnp.maximum(a, b)
                planes = out
            else:
                m = j // _NP
                upper = (li & m) != 0
                nxt = []
                for p in planes:
                    down = _roll(p, _NL - m, axis=1)
                    mn = jnp.minimum(p, down)
                    mx = jnp.maximum(p, down)
                    nxt.append(jnp.where(upper, _roll(mx, m, axis=1), mn))
                planes = nxt
            j //= 2
        planes = flip(planes, k)
        k *= 2
    return planes


def _body(x_ref, y_ref, o_ref):
    xs = _sort_planes([x_ref[:, _NL * p:_NL * (p + 1)] for p in range(_NP)])
    ys = _sort_planes([y_ref[:, _NL * p:_NL * (p + 1)] for p in range(_NP)])
    for p in range(_NP):
        d = ys[p] - xs[p]
        o_ref[p] = 4.0 * jax.nn.sigmoid(2.0 * d) * jax.nn.sigmoid(-2.0 * d)


@functools.partial(jax.jit, static_argnames=("interpret",))
def kernel(x, y, interpret=False):
    b, s, n = x.shape
    rows = b * s
    xf = x.reshape(rows, n)
    yf = y.reshape(rows, n)
    out = pl.pallas_call(
        _body,
        grid=(rows // _BR,),
        in_specs=[
            pl.BlockSpec((_BR, n), lambda i: (i, 0)),
            pl.BlockSpec((_BR, n), lambda i: (i, 0)),
        ],
        out_specs=pl.BlockSpec((_NP, _BR, _NL), lambda i: (0, i, 0)),
        out_shape=jax.ShapeDtypeStruct((_NP, rows, _NL), jnp.float32),
        interpret=interpret,
    )(xf, yf)
    return out.transpose(1, 2, 0).reshape(b, s, n)
